# Initial kernel scaffold; baseline (speedup 1.0000x reference)
import jax
import jax.numpy as jnp
from jax.experimental import pallas as pl


def kernel(x, seq_lens, emb, W1, b1, W2, b2):
    raise NotImplementedError("write your pallas kernel here")


# speedup vs baseline: 2.7676x; 2.7676x over previous
import functools

import jax
import jax.numpy as jnp
from jax import lax
from jax.experimental import pallas as pl
from jax.experimental.pallas import tpu as pltpu
from jax.experimental.pallas import tpu_sc as plsc

_B = 16384
_L = 200
_D = 64
_NW = 32
_EPW = _B // _NW
_G = 64
_NG = _EPW // _G
_C0 = 128
_C1 = _L - _C0


def _sc_pool_body(xf_hbm, emb_hbm, out_hbm, idx_v, rows_v, pooled_v, sem0, sem1):
    wid = lax.axis_index("s") * 2 + lax.axis_index("c")
    sems = (sem0, sem1)

    def fire(el, b):
        base = pl.multiple_of(el * _L, 8)
        rb = rows_v.at[b]
        pltpu.async_copy(emb_hbm.at[idx_v.at[pl.ds(base, _C0)]],
                         rb.at[pl.ds(0, _C0)], sems[b])
        pltpu.async_copy(emb_hbm.at[idx_v.at[pl.ds(base + _C0, _C1)]],
                         rb.at[pl.ds(_C0, _C1)], sems[b])

    def wait(b):
        pltpu.make_async_copy(emb_hbm.at[pl.ds(0, _L)], rows_v.at[b], sems[b]).wait()

    def accumulate(el, b):
        def rbody(i, acc):
            acc = list(acc)
            for k in range(8):
                r = i * 8 + k
                p = k % 2
                for j in range(4):
                    acc[j * 2 + p] = acc[j * 2 + p] + rows_v[b, r, pl.ds(j * 16, 16)]
            return tuple(acc)

        zero = jnp.zeros((16,), jnp.float32)
        acc = lax.fori_loop(0, _L // 8, rbody, (zero,) * 8)
        for j in range(4):
            pooled_v[el, pl.ds(j * 16, 16)] = acc[j * 2] + acc[j * 2 + 1]

    def group_body(g, _):
        row0 = pl.multiple_of(wid * _EPW + g * _G, 8)
        pltpu.sync_copy(xf_hbm.at[pl.ds(pl.multiple_of(row0 * _L, 8), _G * _L)], idx_v)
        fire(0, 0)

        def pair_body(t, __):
            for b in range(2):
                el = t * 2 + b

                @pl.when(el + 1 < _G)
                def _():
                    fire(el + 1, b ^ 1)

                wait(b)
                accumulate(el, b)
            return 0

        lax.fori_loop(0, _G // 2, pair_body, 0)
        pltpu.sync_copy(pooled_v, out_hbm.at[pl.ds(row0, _G), :])
        return 0

    lax.fori_loop(0, _NG, group_body, 0)


def _sc_pool(x_flat, emb):
    mesh = plsc.VectorSubcoreMesh(core_axis_name="c", subcore_axis_name="s")
    f = pl.kernel(
        _sc_pool_body,
        out_type=jax.ShapeDtypeStruct((_B, _D), jnp.float32),
        mesh=mesh,
        scratch_types=[
            pltpu.VMEM((_G * _L,), jnp.int32),
            pltpu.VMEM((2, _L, _D), jnp.float32),
            pltpu.VMEM((_G, _D), jnp.float32),
            pltpu.SemaphoreType.DMA,
            pltpu.SemaphoreType.DMA,
        ],
        compiler_params=pltpu.CompilerParams(use_tc_tiling_on_sc=False),
    )
    return f(x_flat, emb)


def _mlp_body(p_ref, w1_ref, b1_ref, w2_ref, b2_ref, o_ref):
    p = p_ref[...] * (1.0 / _L)
    h = lax.dot_general(p, w1_ref[...], (((1,), (1,)), ((), ())),
                        preferred_element_type=jnp.float32) + b1_ref[...]
    h = jnp.maximum(h, 0.0)
    o_ref[...] = lax.dot_general(h, w2_ref[...], (((1,), (1,)), ((), ())),
                                 preferred_element_type=jnp.float32) + b2_ref[...]


def _tc_mlp(pooled, W1, b1, W2, b2):
    bt = 2048
    nc = W2.shape[0]
    return pl.pallas_call(
        _mlp_body,
        grid=(_B // bt,),
        in_specs=[
            pl.BlockSpec((bt, _D), lambda i: (i, 0)),
            pl.BlockSpec((_D, _D), lambda i: (0, 0)),
            pl.BlockSpec((1, _D), lambda i: (0, 0)),
            pl.BlockSpec((nc, _D), lambda i: (0, 0)),
            pl.BlockSpec((1, nc), lambda i: (0, 0)),
        ],
        out_specs=pl.BlockSpec((bt, nc), lambda i: (i, 0)),
        out_shape=jax.ShapeDtypeStruct((_B, nc), jnp.float32),
    )(pooled, W1, b1, W2, b2)


def kernel(x, seq_lens, emb, W1, b1, W2, b2):
    del seq_lens
    pooled = _sc_pool(x.reshape(-1), emb)
    return _tc_mlp(pooled, W1, b1.reshape(1, -1), W2, b2.reshape(1, -1))


# speedup vs baseline: 3.5316x; 1.2760x over previous
import jax
import jax.numpy as jnp
from jax import lax
from jax.experimental import pallas as pl
from jax.experimental.pallas import tpu as pltpu
from jax.experimental.pallas import tpu_sc as plsc

_B = 16384
_L = 200
_D = 64
_NW = 32
_EPW = _B // _NW
_G = 64
_NG = _EPW // _G
_C0 = 128
_C1 = _L - _C0


def _sc_pool_body(xf_hbm, emb_hbm, out_hbm, idx_v, rows_v, pooled_v, sem0, sem1):
    wid = lax.axis_index("s") * 2 + lax.axis_index("c")
    sems = (sem0, sem1)

    def fire(el, b):
        base = pl.multiple_of(el * _L, 8)
        rb = rows_v.at[b]
        pltpu.async_copy(emb_hbm.at[idx_v.at[pl.ds(base, _C0)]],
                         rb.at[pl.ds(0, _C0)], sems[b])
        pltpu.async_copy(emb_hbm.at[idx_v.at[pl.ds(base + _C0, _C1)]],
                         rb.at[pl.ds(_C0, _C1)], sems[b])

    def wait(b):
        pltpu.make_async_copy(emb_hbm.at[pl.ds(0, _L)], rows_v.at[b], sems[b]).wait()

    def accumulate(el, b):
        def rbody(i, acc):
            acc = list(acc)
            for k in range(8):
                r = i * 8 + k
                p = k % 2
                for j in range(4):
                    acc[j * 2 + p] = acc[j * 2 + p] + rows_v[b, r, pl.ds(j * 16, 16)]
            return tuple(acc)

        zero = jnp.zeros((16,), jnp.float32)
        acc = lax.fori_loop(0, _L // 8, rbody, (zero,) * 8)
        for j in range(4):
            pooled_v[el, pl.ds(j * 16, 16)] = acc[j * 2] + acc[j * 2 + 1]

    def group_body(g, _):
        row0 = pl.multiple_of(wid * _EPW + g * _G, 8)
        pltpu.sync_copy(xf_hbm.at[pl.ds(pl.multiple_of(row0 * _L, 8), _G * _L)], idx_v)
        fire(0, 0)

        def pair_body(t, __):
            for b in range(2):
                el = t * 2 + b

                @pl.when(el + 1 < _G)
                def _():
                    fire(el + 1, b ^ 1)

                wait(b)
                accumulate(el, b)
            return 0

        lax.fori_loop(0, _G // 2, pair_body, 0)
        pltpu.sync_copy(pooled_v, out_hbm.at[pl.ds(row0, _G), :])
        return 0

    lax.fori_loop(0, _NG, group_body, 0)


def _sc_pool(x_flat, emb):
    mesh = plsc.VectorSubcoreMesh(core_axis_name="c", subcore_axis_name="s")
    f = pl.kernel(
        _sc_pool_body,
        out_type=jax.ShapeDtypeStruct((_B, _D), jnp.float32),
        mesh=mesh,
        scratch_types=[
            pltpu.VMEM((_G * _L,), jnp.int32),
            pltpu.VMEM((2, _L, _D), jnp.float32),
            pltpu.VMEM((_G, _D), jnp.float32),
            pltpu.SemaphoreType.DMA,
            pltpu.SemaphoreType.DMA,
        ],
        compiler_params=pltpu.CompilerParams(use_tc_tiling_on_sc=False),
    )
    return f(x_flat, emb)


_TRW = 2048


def _tr_body(e_ref, y_ref):
    c = jnp.concatenate([e_ref[:, : _TRW // 2], e_ref[:, _TRW // 2:]], axis=0)
    y_ref[...] = c.T


def _tc_relayout(emb_t):
    v = emb_t.shape[1]
    g = (v + _TRW - 1) // _TRW
    return pl.pallas_call(
        _tr_body,
        grid=(g,),
        in_specs=[pl.BlockSpec((_D, _TRW), lambda i: (0, i))],
        out_specs=pl.BlockSpec((_TRW // 2, 2 * _D), lambda i: (i, 0)),
        out_shape=jax.ShapeDtypeStruct((g * (_TRW // 2), 2 * _D), jnp.float32),
    )(emb_t)


def _remap_idx(x):
    return (x & ~2047) + ((x & 1023) << 1) + ((x >> 10) & 1)


def _mlp_body(p_ref, w1_ref, b1_ref, w2_ref, b2_ref, ot_ref):
    p = p_ref[...] * (1.0 / _L)
    h = lax.dot_general(p, w1_ref[...], (((1,), (1,)), ((), ())),
                        preferred_element_type=jnp.float32) + b1_ref[...]
    h = jnp.maximum(h, 0.0)
    ot_ref[...] = lax.dot_general(w2_ref[...], h, (((1,), (1,)), ((), ())),
                                  preferred_element_type=jnp.float32) + b2_ref[...]


def _tc_mlp_t(pooled, W1, b1, W2, b2):
    bt = 2048
    nc = W2.shape[0]
    return pl.pallas_call(
        _mlp_body,
        grid=(_B // bt,),
        in_specs=[
            pl.BlockSpec((bt, _D), lambda i: (i, 0)),
            pl.BlockSpec((_D, _D), lambda i: (0, 0)),
            pl.BlockSpec((1, _D), lambda i: (0, 0)),
            pl.BlockSpec((nc, _D), lambda i: (0, 0)),
            pl.BlockSpec((nc, 1), lambda i: (0, 0)),
        ],
        out_specs=pl.BlockSpec((nc, bt), lambda i: (0, i)),
        out_shape=jax.ShapeDtypeStruct((nc, _B), jnp.float32),
    )(pooled, W1, b1, W2, b2)


def kernel(x, seq_lens, emb, W1, b1, W2, b2):
    del seq_lens
    x_flat = lax.optimization_barrier(_remap_idx(x).reshape(-1))
    y = _tc_relayout(emb.T)
    emb_lin = y.reshape(y.shape[0] * 2, _D)
    pooled = _sc_pool(x_flat, emb_lin)
    out_t = _tc_mlp_t(pooled, W1, b1.reshape(1, -1), W2, b2.reshape(-1, 1))
    return out_t.T
